# Initial kernel scaffold; baseline (speedup 1.0000x reference)
import jax
import jax.numpy as jnp
from jax.experimental import pallas as pl


def kernel(x, edge_index, pred_label, node_feat_mask, edge_mask, W):
    raise NotImplementedError("write your pallas kernel here")


# speedup vs baseline: 31.8640x; 31.8640x over previous
import functools

import jax
import jax.numpy as jnp
from jax import lax
from jax.experimental import pallas as pl
from jax.experimental.pallas import tpu as pltpu
from jax.experimental.pallas import tpu_sc as plsc

_EPS = 1e-15
_NC = 2
_NS = 16
_NW = _NC * _NS
_L = 16
_LN2 = 0.6931471805599453
_SQRT2 = 1.4142135381698608


def _vlog(v):
    bits = lax.bitcast_convert_type(v, jnp.int32)
    k = jnp.right_shift(bits, 23) - 127
    mb = jnp.bitwise_or(jnp.bitwise_and(bits, 0x007FFFFF), 0x3F800000)
    m = lax.bitcast_convert_type(mb, jnp.float32)
    big = m > _SQRT2
    m = jnp.where(big, m * 0.5, m)
    kf = k.astype(jnp.float32) + jnp.where(big, 1.0, 0.0)
    s = (m - 1.0) / (m + 1.0)
    s2 = s * s
    poly = 1.0 + s2 * (1.0 / 3.0 + s2 * (0.2 + s2 * (1.0 / 7.0 + s2 * (1.0 / 9.0))))
    return kf * _LN2 + 2.0 * s * poly


def _sigmoid_terms(z):
    zc = jnp.clip(z, -80.0, 80.0)
    t = jnp.exp(-zc)
    u = 1.0 + t
    p = 1.0 / u
    logu = _vlog(u)
    ent = logu + (t * p) * zc
    return p, ent


def _stage1_body(em_hbm, src_hbm, dst_hbm, x_hbm, s1_hbm, s2_hbm, g_hbm,
                 emv, dstv, srcv, hb, wb, row1, accr, stage, tmpi, sem,
                 *, ch, d):
    wid = lax.axis_index("c") * _NS + lax.axis_index("s")
    base = wid * ch
    nvec = ch // _L
    ndg = d // _L
    cap = nvec

    pltpu.sync_copy(em_hbm.at[pl.ds(base, ch)], emv)
    pltpu.sync_copy(dst_hbm.at[pl.ds(base, ch)], dstv)
    pltpu.sync_copy(src_hbm.at[pl.ds(base, ch)], srcv)

    for k in range(ndg):
        accr[pl.ds(k * _L, _L)] = jnp.zeros((_L,), jnp.float32)

    zf = jnp.zeros((_L,), jnp.float32)
    iota16 = lax.broadcasted_iota(jnp.int32, (_L,), 0)
    lane_base = iota16 * cap

    def body(i, carry):
        cnt, s1v, s2v = carry
        off = i * _L
        z = emv[pl.ds(off, _L)]
        p, ent = _sigmoid_terms(z)
        m = dstv[pl.ds(off, _L)] == 0
        sv = srcv[pl.ds(off, _L)]
        slot = lane_base + cnt
        plsc.store_scatter(hb, [slot], sv.astype(jnp.float32), mask=m)
        plsc.store_scatter(wb, [slot], p, mask=m)
        cnt = cnt + jnp.where(m, 1, 0)
        return cnt, s1v + p, s2v + ent

    zi = jnp.zeros((_L,), jnp.int32)
    cnt, s1v, s2v = lax.fori_loop(0, nvec, body, (zi, zf, zf))

    def dbody_for(l):
        def dbody(t, c):
            pos = jnp.full((_L,), l * cap + t, jnp.int32)
            tmpi[...] = plsc.load_gather(hb, [pos]).astype(jnp.int32)
            pltpu.async_copy(x_hbm.at[tmpi.at[pl.ds(0, 1)]], row1, sem).wait()
            wv = plsc.load_gather(wb, [pos])
            for k in range(ndg):
                plsc.addupdate(accr.at[pl.ds(k * _L, _L)],
                               wv * row1[0, pl.ds(k * _L, _L)])
            return c
        return dbody

    for l in range(_L):
        lax.fori_loop(0, cnt[l], dbody_for(l), jnp.int32(0))

    stage[...] = s1v
    pltpu.sync_copy(stage, s1_hbm.at[wid])
    stage[...] = s2v
    pltpu.sync_copy(stage, s2_hbm.at[wid])
    pltpu.sync_copy(accr, g_hbm.at[wid])


def kernel(x, edge_index, pred_label, node_feat_mask, edge_mask, W):
    e = edge_mask.shape[0]
    d = x.shape[1]
    ch = e // _NW

    mesh = plsc.VectorSubcoreMesh(core_axis_name="c", subcore_axis_name="s",
                                  num_cores=_NC, num_subcores=_NS)
    f32 = jnp.float32

    stage1 = pl.kernel(
        functools.partial(_stage1_body, ch=ch, d=d),
        out_type=(
            jax.ShapeDtypeStruct((_NW, _L), f32),
            jax.ShapeDtypeStruct((_NW, _L), f32),
            jax.ShapeDtypeStruct((_NW, d), f32),
        ),
        mesh=mesh,
        compiler_params=pltpu.CompilerParams(needs_layout_passes=False),
        scratch_types=[
            pltpu.VMEM((ch,), f32),
            pltpu.VMEM((ch,), jnp.int32),
            pltpu.VMEM((ch,), jnp.int32),
            pltpu.VMEM((ch,), f32),
            pltpu.VMEM((ch,), f32),
            pltpu.VMEM((1, d), f32),
            pltpu.VMEM((d,), f32),
            pltpu.VMEM((_L,), f32),
            pltpu.VMEM((_L,), jnp.int32),
            pltpu.SemaphoreType.DMA,
        ],
    )
    s1p, s2p, gp = stage1(edge_mask, edge_index[0], edge_index[1], x)

    fm = jax.nn.sigmoid(node_feat_mask)
    g = gp.sum(0)
    logits = (g * fm) @ W
    ll = jax.nn.log_softmax(logits)
    loss = -ll[pred_label[0]]
    loss = loss + 0.005 * jnp.sum(s1p)
    loss = loss + jnp.sum(s2p) / e
    loss = loss + fm.sum()
    ent2 = -fm * jnp.log(fm + _EPS) - (1 - fm) * jnp.log(1 - fm + _EPS)
    loss = loss + 0.1 * ent2.mean()
    return loss


# speedup vs baseline: 32.2077x; 1.0108x over previous
import functools

import jax
import jax.numpy as jnp
from jax import lax
from jax.experimental import pallas as pl
from jax.experimental.pallas import tpu as pltpu
from jax.experimental.pallas import tpu_sc as plsc

_NC = 2
_NS = 16
_NW = _NC * _NS
_L = 16
_LN2 = 0.6931471805599453
_UNROLL = 5

_LC = (1.4720650106103683e-06, 0.9998476974962455, -0.49737321615804786,
       0.31574731675834483, -0.19035433673374297, 0.0826912371119928,
       -0.01741407752444414)


def _vlog(v):
    bits = lax.bitcast_convert_type(v, jnp.int32)
    k = jnp.right_shift(bits, 23) - 127
    mb = jnp.bitwise_or(jnp.bitwise_and(bits, 0x007FFFFF), 0x3F800000)
    f = lax.bitcast_convert_type(mb, jnp.float32) - 1.0
    p = _LC[6]
    for c in (_LC[5], _LC[4], _LC[3], _LC[2], _LC[1], _LC[0]):
        p = p * f + c
    return k.astype(jnp.float32) * _LN2 + p


def _sigmoid_terms(z):
    zc = jnp.clip(z, -80.0, 80.0)
    t = jnp.exp(-zc)
    u = 1.0 + t
    p = 1.0 / u
    ent = _vlog(u) + (1.0 - p) * zc
    return p, ent


def _stage1_body(em_hbm, src_hbm, dst_hbm, x_hbm, s1_hbm, s2_hbm, g_hbm,
                 emv, dstv, srcv, hb, wb, row1, accr, stage, tmpi, sem,
                 *, ch, d):
    wid = lax.axis_index("c") * _NS + lax.axis_index("s")
    base = wid * ch
    nvec = ch // _L
    ndg = d // _L
    cap = nvec

    pltpu.sync_copy(em_hbm.at[pl.ds(base, ch)], emv)
    pltpu.sync_copy(dst_hbm.at[pl.ds(base, ch)], dstv)
    pltpu.sync_copy(src_hbm.at[pl.ds(base, ch)], srcv)

    for k in range(ndg):
        accr[pl.ds(k * _L, _L)] = jnp.zeros((_L,), jnp.float32)

    zf = jnp.zeros((_L,), jnp.float32)
    zi = jnp.zeros((_L,), jnp.int32)
    iota16 = lax.broadcasted_iota(jnp.int32, (_L,), 0)
    lane_base = iota16 * cap

    nun = nvec // _UNROLL

    def body(i, carry):
        cnt, s1s, s2s = carry
        s1s = list(s1s)
        s2s = list(s2s)
        for uu in range(_UNROLL):
            off = (i * _UNROLL + uu) * _L
            z = emv[pl.ds(off, _L)]
            p, ent = _sigmoid_terms(z)
            m = dstv[pl.ds(off, _L)] == 0
            sv = srcv[pl.ds(off, _L)]
            slot = lane_base + cnt
            plsc.store_scatter(hb, [slot], sv.astype(jnp.float32), mask=m)
            plsc.store_scatter(wb, [slot], p, mask=m)
            cnt = cnt + jnp.where(m, 1, 0)
            s1s[uu] = s1s[uu] + p
            s2s[uu] = s2s[uu] + ent
        return cnt, tuple(s1s), tuple(s2s)

    cnt, s1s, s2s = lax.fori_loop(
        0, nun, body, (zi, (zf,) * _UNROLL, (zf,) * _UNROLL))
    s1v = s1s[0] + s1s[1] + s1s[2] + s1s[3] + s1s[4]
    s2v = s2s[0] + s2s[1] + s2s[2] + s2s[3] + s2s[4]

    def dbody_for(l):
        def dbody(t, c):
            pos = jnp.full((_L,), l * cap + t, jnp.int32)
            tmpi[...] = plsc.load_gather(hb, [pos]).astype(jnp.int32)
            pltpu.async_copy(x_hbm.at[tmpi.at[pl.ds(0, 1)]], row1, sem).wait()
            wv = plsc.load_gather(wb, [pos])
            for k in range(ndg):
                plsc.addupdate(accr.at[pl.ds(k * _L, _L)],
                               wv * row1[0, pl.ds(k * _L, _L)])
            return c
        return dbody

    for l in range(_L):
        lax.fori_loop(0, cnt[l], dbody_for(l), jnp.int32(0))

    stage[...] = s1v
    pltpu.sync_copy(stage, s1_hbm.at[wid])
    stage[...] = s2v
    pltpu.sync_copy(stage, s2_hbm.at[wid])
    pltpu.sync_copy(accr, g_hbm.at[wid])


def _stage2_body(s1_hbm, s2_hbm, g_hbm, nfm_hbm, w_hbm, pl_hbm, out_hbm,
                 b_s1, b_s2, b_g, b_nfm, b_w, b_pl, stage, sem,
                 *, e, d):
    wid = lax.axis_index("c") * _NS + lax.axis_index("s")
    ndg = d // _L

    @pl.when(wid == 0)
    def _():
        pltpu.sync_copy(s1_hbm, b_s1)
        pltpu.sync_copy(s2_hbm, b_s2)
        pltpu.sync_copy(g_hbm, b_g)
        pltpu.sync_copy(nfm_hbm, b_nfm)
        pltpu.sync_copy(w_hbm, b_w)
        pltpu.sync_copy(pl_hbm.at[pl.ds(0, _L)], b_pl)

        zf = jnp.zeros((_L,), jnp.float32)
        s1v, s2v = zf, zf
        for w in range(_NW):
            s1v = s1v + b_s1[w]
            s2v = s2v + b_s2[w]
        s1 = jnp.sum(s1v, axis=0)
        s2 = jnp.sum(s2v, axis=0)

        fmsum = zf
        ent2v = zf
        logits = zf
        for k in range(ndg):
            gk = zf
            for w in range(_NW):
                gk = gk + b_g[w, pl.ds(k * _L, _L)]
            zk = b_nfm[pl.ds(k * _L, _L)]
            fmk, entk = _sigmoid_terms(zk)
            fmsum = fmsum + fmk
            ent2v = ent2v + entk
            aggk = gk * fmk
            for j in range(_L):
                logits = logits + aggk[j] * b_w[k * _L + j]

        mx = jnp.max(logits, axis=0)
        sh = logits - mx
        se = jnp.sum(jnp.exp(sh), axis=0)
        logsm = sh - _vlog(jnp.full((_L,), se))
        lab = b_pl[...][0]
        ii = lax.broadcasted_iota(jnp.int32, (_L,), 0)
        pick = jnp.sum(jnp.where(ii == lab, logsm, 0.0), axis=0)

        loss = (-pick
                + 0.005 * s1
                + s2 * (1.0 / float(e))
                + jnp.sum(fmsum, axis=0)
                + jnp.sum(ent2v, axis=0) * (0.1 / float(d)))
        stage[...] = jnp.full((_L,), loss)
        pltpu.sync_copy(stage, out_hbm)


def kernel(x, edge_index, pred_label, node_feat_mask, edge_mask, W):
    e = edge_mask.shape[0]
    d = x.shape[1]
    c = W.shape[1]
    ch = e // _NW

    mesh = plsc.VectorSubcoreMesh(core_axis_name="c", subcore_axis_name="s",
                                  num_cores=_NC, num_subcores=_NS)
    f32 = jnp.float32
    params = pltpu.CompilerParams(needs_layout_passes=False)

    stage1 = pl.kernel(
        functools.partial(_stage1_body, ch=ch, d=d),
        out_type=(
            jax.ShapeDtypeStruct((_NW, _L), f32),
            jax.ShapeDtypeStruct((_NW, _L), f32),
            jax.ShapeDtypeStruct((_NW, d), f32),
        ),
        mesh=mesh,
        compiler_params=params,
        scratch_types=[
            pltpu.VMEM((ch,), f32),
            pltpu.VMEM((ch,), jnp.int32),
            pltpu.VMEM((ch,), jnp.int32),
            pltpu.VMEM((ch,), f32),
            pltpu.VMEM((ch,), f32),
            pltpu.VMEM((1, d), f32),
            pltpu.VMEM((d,), f32),
            pltpu.VMEM((_L,), f32),
            pltpu.VMEM((_L,), jnp.int32),
            pltpu.SemaphoreType.DMA,
        ],
    )
    s1p, s2p, gp = stage1(edge_mask, edge_index[0], edge_index[1], x)

    stage2 = pl.kernel(
        functools.partial(_stage2_body, e=e, d=d),
        out_type=jax.ShapeDtypeStruct((_L,), f32),
        mesh=mesh,
        compiler_params=params,
        scratch_types=[
            pltpu.VMEM((_NW, _L), f32),
            pltpu.VMEM((_NW, _L), f32),
            pltpu.VMEM((_NW, d), f32),
            pltpu.VMEM((d,), f32),
            pltpu.VMEM((d, c), f32),
            pltpu.VMEM((_L,), jnp.int32),
            pltpu.VMEM((_L,), f32),
            pltpu.SemaphoreType.DMA,
        ],
    )
    out = stage2(s1p, s2p, gp, node_feat_mask, W, pred_label)
    return out[0]


# speedup vs baseline: 34.4417x; 1.0694x over previous
import functools

import jax
import jax.numpy as jnp
from jax import lax
from jax.experimental import pallas as pl
from jax.experimental.pallas import tpu as pltpu
from jax.experimental.pallas import tpu_sc as plsc

_NC = 2
_NS = 16
_NW = _NC * _NS
_L = 16
_LN2 = 0.6931471805599453
_UNROLL = 5

_LC = (1.4720650106103683e-06, 0.9998476974962455, -0.49737321615804786,
       0.31574731675834483, -0.19035433673374297, 0.0826912371119928,
       -0.01741407752444414)


def _vlog(v):
    bits = lax.bitcast_convert_type(v, jnp.int32)
    k = jnp.right_shift(bits, 23) - 127
    mb = jnp.bitwise_or(jnp.bitwise_and(bits, 0x007FFFFF), 0x3F800000)
    f = lax.bitcast_convert_type(mb, jnp.float32) - 1.0
    p = _LC[6]
    for c in (_LC[5], _LC[4], _LC[3], _LC[2], _LC[1], _LC[0]):
        p = p * f + c
    return k.astype(jnp.float32) * _LN2 + p


def _sigmoid_terms(z):
    zc = jnp.clip(z, -80.0, 80.0)
    t = jnp.exp(-zc)
    u = 1.0 + t
    p = 1.0 / u
    ent = _vlog(u) + (1.0 - p) * zc
    return p, ent


def _sigmoid_only(z):
    zc = jnp.maximum(z, -80.0)
    return 1.0 / (1.0 + jnp.exp(-zc))


def _stage1_body(em_hbm, src_hbm, dst_hbm, x_hbm, nfm_hbm, w_hbm,
                 s1_hbm, s2_hbm, lg_hbm,
                 emv, dstv, srcv, hb, nfmv, wv, row1, accr, stage, tmpi, sem,
                 *, ch, d):
    wid = lax.axis_index("c") * _NS + lax.axis_index("s")
    base = wid * ch
    nvec = ch // _L
    ndg = d // _L
    cap = nvec

    pltpu.sync_copy(em_hbm.at[pl.ds(base, ch)], emv)
    pltpu.sync_copy(dst_hbm.at[pl.ds(base, ch)], dstv)
    pltpu.sync_copy(src_hbm.at[pl.ds(base, ch)], srcv)
    pltpu.sync_copy(nfm_hbm, nfmv)
    pltpu.sync_copy(w_hbm, wv)

    for k in range(ndg):
        accr[pl.ds(k * _L, _L)] = jnp.zeros((_L,), jnp.float32)

    zf = jnp.zeros((_L,), jnp.float32)
    zi = jnp.zeros((_L,), jnp.int32)
    iota16 = lax.broadcasted_iota(jnp.int32, (_L,), 0)
    lane_base = iota16 * cap

    nun = nvec // _UNROLL

    def body(i, carry):
        cnt, s1s, s2s = carry
        s1s = list(s1s)
        s2s = list(s2s)
        for uu in range(_UNROLL):
            off = (i * _UNROLL + uu) * _L
            z = emv[pl.ds(off, _L)]
            p, ent = _sigmoid_terms(z)
            m = dstv[pl.ds(off, _L)] == 0
            slot = lane_base + cnt
            ev = (off + iota16).astype(jnp.float32)
            plsc.store_scatter(hb, [slot], ev, mask=m)
            cnt = cnt + jnp.where(m, 1, 0)
            s1s[uu] = s1s[uu] + p
            s2s[uu] = s2s[uu] + ent
        return cnt, tuple(s1s), tuple(s2s)

    cnt, s1s, s2s = lax.fori_loop(
        0, nun, body, (zi, (zf,) * _UNROLL, (zf,) * _UNROLL))
    s1v = s1s[0] + s1s[1] + s1s[2] + s1s[3] + s1s[4]
    s2v = s2s[0] + s2s[1] + s2s[2] + s2s[3] + s2s[4]

    def dbody_for(l):
        def dbody(t, c):
            pos = jnp.full((_L,), l * cap + t, jnp.int32)
            ev = plsc.load_gather(hb, [pos]).astype(jnp.int32)
            tmpi[...] = plsc.load_gather(srcv, [ev])
            zv = plsc.load_gather(emv, [ev])
            pltpu.async_copy(x_hbm.at[tmpi.at[pl.ds(0, 1)]], row1, sem).wait()
            wvv = _sigmoid_only(zv)
            for k in range(ndg):
                plsc.addupdate(accr.at[pl.ds(k * _L, _L)],
                               wvv * row1[0, pl.ds(k * _L, _L)])
            return c
        return dbody

    for l in range(_L):
        lax.fori_loop(0, cnt[l], dbody_for(l), jnp.int32(0))

    logits = zf
    for k in range(ndg):
        fmk = _sigmoid_only(nfmv[pl.ds(k * _L, _L)])
        aggk = accr[pl.ds(k * _L, _L)] * fmk
        for j in range(_L):
            logits = logits + aggk[j] * wv[k * _L + j]

    stage[...] = s1v
    pltpu.sync_copy(stage, s1_hbm.at[wid])
    stage[...] = s2v
    pltpu.sync_copy(stage, s2_hbm.at[wid])
    stage[...] = logits
    pltpu.sync_copy(stage, lg_hbm.at[wid])


def _stage2_body(s1_hbm, s2_hbm, lg_hbm, nfm_hbm, pl_hbm, out_hbm,
                 b_s1, b_s2, b_lg, b_nfm, b_pl, stage, sem,
                 *, e, d):
    wid = lax.axis_index("c") * _NS + lax.axis_index("s")
    ndg = d // _L

    @pl.when(wid == 0)
    def _():
        pltpu.sync_copy(s1_hbm, b_s1)
        pltpu.sync_copy(s2_hbm, b_s2)
        pltpu.sync_copy(lg_hbm, b_lg)
        pltpu.sync_copy(nfm_hbm, b_nfm)
        pltpu.sync_copy(pl_hbm.at[pl.ds(0, _L)], b_pl)

        zf = jnp.zeros((_L,), jnp.float32)
        s1v, s2v, logits = zf, zf, zf
        for w in range(_NW):
            s1v = s1v + b_s1[w]
            s2v = s2v + b_s2[w]
            logits = logits + b_lg[w]
        s1 = jnp.sum(s1v, axis=0)
        s2 = jnp.sum(s2v, axis=0)

        fmsum = zf
        ent2v = zf
        for k in range(ndg):
            zk = b_nfm[pl.ds(k * _L, _L)]
            fmk, entk = _sigmoid_terms(zk)
            fmsum = fmsum + fmk
            ent2v = ent2v + entk

        mx = jnp.max(logits, axis=0)
        sh = logits - mx
        se = jnp.sum(jnp.exp(sh), axis=0)
        logsm = sh - _vlog(jnp.full((_L,), se))
        lab = b_pl[...][0]
        ii = lax.broadcasted_iota(jnp.int32, (_L,), 0)
        pick = jnp.sum(jnp.where(ii == lab, logsm, 0.0), axis=0)

        loss = (-pick
                + 0.005 * s1
                + s2 * (1.0 / float(e))
                + jnp.sum(fmsum, axis=0)
                + jnp.sum(ent2v, axis=0) * (0.1 / float(d)))
        stage[...] = jnp.full((_L,), loss)
        pltpu.sync_copy(stage, out_hbm)


def kernel(x, edge_index, pred_label, node_feat_mask, edge_mask, W):
    e = edge_mask.shape[0]
    d = x.shape[1]
    c = W.shape[1]
    ch = e // _NW

    mesh = plsc.VectorSubcoreMesh(core_axis_name="c", subcore_axis_name="s",
                                  num_cores=_NC, num_subcores=_NS)
    f32 = jnp.float32
    params = pltpu.CompilerParams(needs_layout_passes=False)

    stage1 = pl.kernel(
        functools.partial(_stage1_body, ch=ch, d=d),
        out_type=(
            jax.ShapeDtypeStruct((_NW, _L), f32),
            jax.ShapeDtypeStruct((_NW, _L), f32),
            jax.ShapeDtypeStruct((_NW, _L), f32),
        ),
        mesh=mesh,
        compiler_params=params,
        scratch_types=[
            pltpu.VMEM((ch,), f32),
            pltpu.VMEM((ch,), jnp.int32),
            pltpu.VMEM((ch,), jnp.int32),
            pltpu.VMEM((ch,), f32),
            pltpu.VMEM((d,), f32),
            pltpu.VMEM((d, c), f32),
            pltpu.VMEM((1, d), f32),
            pltpu.VMEM((d,), f32),
            pltpu.VMEM((_L,), f32),
            pltpu.VMEM((_L,), jnp.int32),
            pltpu.SemaphoreType.DMA,
        ],
    )
    s1p, s2p, lgp = stage1(edge_mask, edge_index[0], edge_index[1], x,
                           node_feat_mask, W)

    stage2 = pl.kernel(
        functools.partial(_stage2_body, e=e, d=d),
        out_type=jax.ShapeDtypeStruct((_L,), f32),
        mesh=mesh,
        compiler_params=params,
        scratch_types=[
            pltpu.VMEM((_NW, _L), f32),
            pltpu.VMEM((_NW, _L), f32),
            pltpu.VMEM((_NW, _L), f32),
            pltpu.VMEM((d,), f32),
            pltpu.VMEM((_L,), jnp.int32),
            pltpu.VMEM((_L,), f32),
            pltpu.SemaphoreType.DMA,
        ],
    )
    out = stage2(s1p, s2p, lgp, node_feat_mask, pred_label)
    return out[0]


# speedup vs baseline: 35.0719x; 1.0183x over previous
import functools

import jax
import jax.numpy as jnp
from jax import lax
from jax.experimental import pallas as pl
from jax.experimental.pallas import tpu as pltpu
from jax.experimental.pallas import tpu_sc as plsc

_NC = 2
_NS = 16
_NW = _NC * _NS
_L = 16
_LN2 = 0.6931471805599453
_UNROLL = 5

_LC = (1.4720650106103683e-06, 0.9998476974962455, -0.49737321615804786,
       0.31574731675834483, -0.19035433673374297, 0.0826912371119928,
       -0.01741407752444414)


def _vlog(v):
    bits = lax.bitcast_convert_type(v, jnp.int32)
    k = jnp.right_shift(bits, 23) - 127
    mb = jnp.bitwise_or(jnp.bitwise_and(bits, 0x007FFFFF), 0x3F800000)
    f = lax.bitcast_convert_type(mb, jnp.float32) - 1.0
    p = _LC[6]
    for c in (_LC[5], _LC[4], _LC[3], _LC[2], _LC[1], _LC[0]):
        p = p * f + c
    return k.astype(jnp.float32) * _LN2 + p


def _sigmoid_terms(z):
    a = jnp.abs(z)
    t = jnp.exp(-a)
    q = 1.0 / (1.0 + t)
    pf = _LC[6]
    for c in (_LC[5], _LC[4], _LC[3], _LC[2], _LC[1], _LC[0]):
        pf = pf * t + c
    ent = pf + (1.0 - q) * a
    p = jnp.where(z < 0.0, 1.0 - q, q)
    return p, ent


def _sigmoid_only(z):
    zc = jnp.maximum(z, -80.0)
    return 1.0 / (1.0 + jnp.exp(-zc))


def _stage1_body(em_hbm, src_hbm, dst_hbm, x_hbm, nfm_hbm, w_hbm,
                 s1_hbm, s2_hbm, lg_hbm,
                 emv, dstv, srcv, hb, nfmv, wv, row1, accr, stage, tmpi, sem,
                 *, ch, d):
    wid = lax.axis_index("c") * _NS + lax.axis_index("s")
    base = wid * ch
    nvec = ch // _L
    ndg = d // _L
    cap = nvec

    pltpu.sync_copy(em_hbm.at[pl.ds(base, ch)], emv)
    pltpu.sync_copy(dst_hbm.at[pl.ds(base, ch)], dstv)
    pltpu.sync_copy(src_hbm.at[pl.ds(base, ch)], srcv)
    pltpu.sync_copy(nfm_hbm, nfmv)
    pltpu.sync_copy(w_hbm, wv)

    for k in range(ndg):
        accr[pl.ds(k * _L, _L)] = jnp.zeros((_L,), jnp.float32)

    zf = jnp.zeros((_L,), jnp.float32)
    zi = jnp.zeros((_L,), jnp.int32)
    iota16 = lax.broadcasted_iota(jnp.int32, (_L,), 0)
    lane_base = iota16 * cap

    nun = nvec // _UNROLL

    def body(i, carry):
        cnt, s1s, s2s = carry
        s1s = list(s1s)
        s2s = list(s2s)
        for uu in range(_UNROLL):
            off = (i * _UNROLL + uu) * _L
            z = emv[pl.ds(off, _L)]
            p, ent = _sigmoid_terms(z)
            m = dstv[pl.ds(off, _L)] == 0
            slot = lane_base + cnt
            plsc.store_scatter(hb, [slot], off + iota16, mask=m)
            cnt = cnt + jnp.where(m, 1, 0)
            s1s[uu] = s1s[uu] + p
            s2s[uu] = s2s[uu] + ent
        return cnt, tuple(s1s), tuple(s2s)

    cnt, s1s, s2s = lax.fori_loop(
        0, nun, body, (zi, (zf,) * _UNROLL, (zf,) * _UNROLL))
    s1v = s1s[0] + s1s[1] + s1s[2] + s1s[3] + s1s[4]
    s2v = s2s[0] + s2s[1] + s2s[2] + s2s[3] + s2s[4]

    def dbody_for(l):
        def dbody(t, c):
            pos = jnp.full((_L,), l * cap + t, jnp.int32)
            ev = plsc.load_gather(hb, [pos])
            tmpi[...] = plsc.load_gather(srcv, [ev])
            zv = plsc.load_gather(emv, [ev])
            pltpu.async_copy(x_hbm.at[tmpi.at[pl.ds(0, 1)]], row1, sem).wait()
            wvv = _sigmoid_only(zv)
            for k in range(ndg):
                plsc.addupdate(accr.at[pl.ds(k * _L, _L)],
                               wvv * row1[0, pl.ds(k * _L, _L)])
            return c
        return dbody

    for l in range(_L):
        lax.fori_loop(0, cnt[l], dbody_for(l), jnp.int32(0))

    logits = zf
    for k in range(ndg):
        fmk = _sigmoid_only(nfmv[pl.ds(k * _L, _L)])
        aggk = accr[pl.ds(k * _L, _L)] * fmk
        for j in range(_L):
            logits = logits + aggk[j] * wv[k * _L + j]

    stage[...] = s1v
    pltpu.sync_copy(stage, s1_hbm.at[wid])
    stage[...] = s2v
    pltpu.sync_copy(stage, s2_hbm.at[wid])
    stage[...] = logits
    pltpu.sync_copy(stage, lg_hbm.at[wid])


def _stage2_body(s1_hbm, s2_hbm, lg_hbm, nfm_hbm, pl_hbm, out_hbm,
                 b_s1, b_s2, b_lg, b_nfm, b_pl, stage, sem,
                 *, e, d):
    wid = lax.axis_index("c") * _NS + lax.axis_index("s")
    ndg = d // _L

    @pl.when(wid == 0)
    def _():
        pltpu.sync_copy(s1_hbm, b_s1)
        pltpu.sync_copy(s2_hbm, b_s2)
        pltpu.sync_copy(lg_hbm, b_lg)
        pltpu.sync_copy(nfm_hbm, b_nfm)
        pltpu.sync_copy(pl_hbm.at[pl.ds(0, _L)], b_pl)

        zf = jnp.zeros((_L,), jnp.float32)
        s1v, s2v, logits = zf, zf, zf
        for w in range(_NW):
            s1v = s1v + b_s1[w]
            s2v = s2v + b_s2[w]
            logits = logits + b_lg[w]
        s1 = jnp.sum(s1v, axis=0)
        s2 = jnp.sum(s2v, axis=0)

        fmsum = zf
        ent2v = zf
        for k in range(ndg):
            zk = b_nfm[pl.ds(k * _L, _L)]
            fmk, entk = _sigmoid_terms(zk)
            fmsum = fmsum + fmk
            ent2v = ent2v + entk

        mx = jnp.max(logits, axis=0)
        sh = logits - mx
        se = jnp.sum(jnp.exp(sh), axis=0)
        logsm = sh - _vlog(jnp.full((_L,), se))
        lab = b_pl[...][0]
        ii = lax.broadcasted_iota(jnp.int32, (_L,), 0)
        pick = jnp.sum(jnp.where(ii == lab, logsm, 0.0), axis=0)

        loss = (-pick
                + 0.005 * s1
                + s2 * (1.0 / float(e))
                + jnp.sum(fmsum, axis=0)
                + jnp.sum(ent2v, axis=0) * (0.1 / float(d)))
        stage[...] = jnp.full((_L,), loss)
        pltpu.sync_copy(stage, out_hbm)


def kernel(x, edge_index, pred_label, node_feat_mask, edge_mask, W):
    e = edge_mask.shape[0]
    d = x.shape[1]
    c = W.shape[1]
    ch = e // _NW

    mesh = plsc.VectorSubcoreMesh(core_axis_name="c", subcore_axis_name="s",
                                  num_cores=_NC, num_subcores=_NS)
    f32 = jnp.float32
    params = pltpu.CompilerParams(needs_layout_passes=False)

    stage1 = pl.kernel(
        functools.partial(_stage1_body, ch=ch, d=d),
        out_type=(
            jax.ShapeDtypeStruct((_NW, _L), f32),
            jax.ShapeDtypeStruct((_NW, _L), f32),
            jax.ShapeDtypeStruct((_NW, _L), f32),
        ),
        mesh=mesh,
        compiler_params=params,
        scratch_types=[
            pltpu.VMEM((ch,), f32),
            pltpu.VMEM((ch,), jnp.int32),
            pltpu.VMEM((ch,), jnp.int32),
            pltpu.VMEM((ch,), jnp.int32),
            pltpu.VMEM((d,), f32),
            pltpu.VMEM((d, c), f32),
            pltpu.VMEM((1, d), f32),
            pltpu.VMEM((d,), f32),
            pltpu.VMEM((_L,), f32),
            pltpu.VMEM((_L,), jnp.int32),
            pltpu.SemaphoreType.DMA,
        ],
    )
    s1p, s2p, lgp = stage1(edge_mask, edge_index[0], edge_index[1], x,
                           node_feat_mask, W)

    stage2 = pl.kernel(
        functools.partial(_stage2_body, e=e, d=d),
        out_type=jax.ShapeDtypeStruct((_L,), f32),
        mesh=mesh,
        compiler_params=params,
        scratch_types=[
            pltpu.VMEM((_NW, _L), f32),
            pltpu.VMEM((_NW, _L), f32),
            pltpu.VMEM((_NW, _L), f32),
            pltpu.VMEM((d,), f32),
            pltpu.VMEM((_L,), jnp.int32),
            pltpu.VMEM((_L,), f32),
            pltpu.SemaphoreType.DMA,
        ],
    )
    out = stage2(s1p, s2p, lgp, node_feat_mask, pred_label)
    return out[0]


# speedup vs baseline: 38.1561x; 1.0879x over previous
import functools

import jax
import jax.numpy as jnp
from jax import lax
from jax.experimental import pallas as pl
from jax.experimental.pallas import tpu as pltpu
from jax.experimental.pallas import tpu_sc as plsc

_NC = 2
_NS = 16
_NW = _NC * _NS
_L = 16
_LN2 = 0.6931471805599453
_UNROLL = 5

_LC = (1.4720650106103683e-06, 0.9998476974962455, -0.49737321615804786,
       0.31574731675834483, -0.19035433673374297, 0.0826912371119928,
       -0.01741407752444414)
_LC5 = (2.211703119980868e-05, 0.9990104466294621, -0.48915684720231134,
        0.2833043245174014, -0.13011941539123476, 0.030102625011657738)


def _vlog(v):
    bits = lax.bitcast_convert_type(v, jnp.int32)
    k = jnp.right_shift(bits, 23) - 127
    mb = jnp.bitwise_or(jnp.bitwise_and(bits, 0x007FFFFF), 0x3F800000)
    f = lax.bitcast_convert_type(mb, jnp.float32) - 1.0
    p = _LC[6]
    for c in (_LC[5], _LC[4], _LC[3], _LC[2], _LC[1], _LC[0]):
        p = p * f + c
    return k.astype(jnp.float32) * _LN2 + p


def _sigmoid_terms(z):
    a = jnp.abs(z)
    t = jnp.exp(-a)
    q = 1.0 / (1.0 + t)
    pf = _LC5[5]
    for c in (_LC5[4], _LC5[3], _LC5[2], _LC5[1], _LC5[0]):
        pf = pf * t + c
    ent = pf + (1.0 - q) * a
    p = jnp.where(z < 0.0, 1.0 - q, q)
    return p, ent


def _sigmoid_only(z):
    zc = jnp.maximum(z, -80.0)
    return 1.0 / (1.0 + jnp.exp(-zc))


def _stage1_body(em_hbm, src_hbm, dst_hbm, x_hbm, nfm_hbm, w_hbm, out_hbm,
                 emv, dstv, srcv, hb, nfmv, wv, row1, accr, stage, tmpi,
                 sem, sma, smb, smc, smd, sme,
                 *, ch, d):
    wid = lax.axis_index("c") * _NS + lax.axis_index("s")
    base = wid * ch
    nvec = ch // _L
    ndg = d // _L
    cap = nvec

    c_em = pltpu.async_copy(em_hbm.at[pl.ds(base, ch)], emv, sma)
    c_dst = pltpu.async_copy(dst_hbm.at[pl.ds(base, ch)], dstv, smb)
    c_src = pltpu.async_copy(src_hbm.at[pl.ds(base, ch)], srcv, smc)
    c_nfm = pltpu.async_copy(nfm_hbm, nfmv, smd)
    c_w = pltpu.async_copy(w_hbm, wv, sme)

    for k in range(ndg):
        accr[pl.ds(k * _L, _L)] = jnp.zeros((_L,), jnp.float32)

    zf = jnp.zeros((_L,), jnp.float32)
    zi = jnp.zeros((_L,), jnp.int32)
    iota16 = lax.broadcasted_iota(jnp.int32, (_L,), 0)
    lane_base = iota16 * cap

    nun = nvec // _UNROLL

    c_em.wait()
    c_dst.wait()
    c_src.wait()

    def body(i, carry):
        cnt, s1s, s2s = carry
        s1s = list(s1s)
        s2s = list(s2s)
        for uu in range(_UNROLL):
            off = (i * _UNROLL + uu) * _L
            z = emv[pl.ds(off, _L)]
            p, ent = _sigmoid_terms(z)
            m = dstv[pl.ds(off, _L)] == 0
            slot = lane_base + cnt
            plsc.store_scatter(hb, [slot], off + iota16, mask=m)
            cnt = cnt + jnp.where(m, 1, 0)
            s1s[uu] = s1s[uu] + p
            s2s[uu] = s2s[uu] + ent
        return cnt, tuple(s1s), tuple(s2s)

    cnt, s1s, s2s = lax.fori_loop(
        0, nun, body, (zi, (zf,) * _UNROLL, (zf,) * _UNROLL))
    s1v = s1s[0] + s1s[1] + s1s[2] + s1s[3] + s1s[4]
    s2v = s2s[0] + s2s[1] + s2s[2] + s2s[3] + s2s[4]

    def dbody_for(l):
        def dbody(t, c):
            pos = jnp.full((_L,), l * cap + t, jnp.int32)
            ev = plsc.load_gather(hb, [pos])
            tmpi[...] = plsc.load_gather(srcv, [ev])
            zv = plsc.load_gather(emv, [ev])
            pltpu.async_copy(x_hbm.at[tmpi.at[pl.ds(0, 1)]], row1, sem).wait()
            wvv = _sigmoid_only(zv)
            for k in range(ndg):
                plsc.addupdate(accr.at[pl.ds(k * _L, _L)],
                               wvv * row1[0, pl.ds(k * _L, _L)])
            return c
        return dbody

    for l in range(_L):
        lax.fori_loop(0, cnt[l], dbody_for(l), jnp.int32(0))

    c_nfm.wait()
    c_w.wait()
    logits = zf
    for k in range(ndg):
        fmk = _sigmoid_only(nfmv[pl.ds(k * _L, _L)])
        aggk = accr[pl.ds(k * _L, _L)] * fmk
        for j in range(_L):
            logits = logits + aggk[j] * wv[k * _L + j]

    stage[pl.ds(0, _L)] = s1v
    stage[pl.ds(_L, _L)] = s2v
    stage[pl.ds(2 * _L, _L)] = logits
    pltpu.sync_copy(stage, out_hbm.at[wid])


def _stage2_body(part_hbm, nfm_hbm, pl_hbm, out_hbm,
                 b_part, b_nfm, b_pl, stage, sma, smb, smc,
                 *, e, d):
    wid = lax.axis_index("c") * _NS + lax.axis_index("s")
    ndg = d // _L

    @pl.when(wid == 0)
    def _():
        c_p = pltpu.async_copy(part_hbm, b_part, sma)
        c_n = pltpu.async_copy(nfm_hbm, b_nfm, smb)
        c_l = pltpu.async_copy(pl_hbm.at[pl.ds(0, _L)], b_pl, smc)
        c_p.wait()
        c_n.wait()
        c_l.wait()

        zf = jnp.zeros((_L,), jnp.float32)
        s1v, s2v, logits = zf, zf, zf
        for w in range(_NW):
            s1v = s1v + b_part[w, pl.ds(0, _L)]
            s2v = s2v + b_part[w, pl.ds(_L, _L)]
            logits = logits + b_part[w, pl.ds(2 * _L, _L)]
        s1 = jnp.sum(s1v, axis=0)
        s2 = jnp.sum(s2v, axis=0)

        fmsum = zf
        ent2v = zf
        for k in range(ndg):
            zk = b_nfm[pl.ds(k * _L, _L)]
            fmk, entk = _sigmoid_terms(zk)
            fmsum = fmsum + fmk
            ent2v = ent2v + entk

        mx = jnp.max(logits, axis=0)
        sh = logits - mx
        se = jnp.sum(jnp.exp(sh), axis=0)
        logsm = sh - _vlog(jnp.full((_L,), se))
        lab = b_pl[...][0]
        ii = lax.broadcasted_iota(jnp.int32, (_L,), 0)
        pick = jnp.sum(jnp.where(ii == lab, logsm, 0.0), axis=0)

        loss = (-pick
                + 0.005 * s1
                + s2 * (1.0 / float(e))
                + jnp.sum(fmsum, axis=0)
                + jnp.sum(ent2v, axis=0) * (0.1 / float(d)))
        stage[...] = jnp.full((_L,), loss)
        pltpu.sync_copy(stage, out_hbm)


def kernel(x, edge_index, pred_label, node_feat_mask, edge_mask, W):
    e = edge_mask.shape[0]
    d = x.shape[1]
    c = W.shape[1]
    ch = e // _NW

    mesh = plsc.VectorSubcoreMesh(core_axis_name="c", subcore_axis_name="s",
                                  num_cores=_NC, num_subcores=_NS)
    f32 = jnp.float32
    params = pltpu.CompilerParams(needs_layout_passes=False)

    stage1 = pl.kernel(
        functools.partial(_stage1_body, ch=ch, d=d),
        out_type=jax.ShapeDtypeStruct((_NW, 3 * _L), f32),
        mesh=mesh,
        compiler_params=params,
        scratch_types=[
            pltpu.VMEM((ch,), f32),
            pltpu.VMEM((ch,), jnp.int32),
            pltpu.VMEM((ch,), jnp.int32),
            pltpu.VMEM((ch,), jnp.int32),
            pltpu.VMEM((d,), f32),
            pltpu.VMEM((d, c), f32),
            pltpu.VMEM((1, d), f32),
            pltpu.VMEM((d,), f32),
            pltpu.VMEM((3 * _L,), f32),
            pltpu.VMEM((_L,), jnp.int32),
            pltpu.SemaphoreType.DMA,
            pltpu.SemaphoreType.DMA,
            pltpu.SemaphoreType.DMA,
            pltpu.SemaphoreType.DMA,
            pltpu.SemaphoreType.DMA,
            pltpu.SemaphoreType.DMA,
        ],
    )
    part = stage1(edge_mask, edge_index[0], edge_index[1], x,
                  node_feat_mask, W)

    stage2 = pl.kernel(
        functools.partial(_stage2_body, e=e, d=d),
        out_type=jax.ShapeDtypeStruct((_L,), f32),
        mesh=mesh,
        compiler_params=params,
        scratch_types=[
            pltpu.VMEM((_NW, 3 * _L), f32),
            pltpu.VMEM((d,), f32),
            pltpu.VMEM((_L,), jnp.int32),
            pltpu.VMEM((_L,), f32),
            pltpu.SemaphoreType.DMA,
            pltpu.SemaphoreType.DMA,
            pltpu.SemaphoreType.DMA,
        ],
    )
    out = stage2(part, node_feat_mask, pred_label)
    return out[0]


# speedup vs baseline: 41.2403x; 1.0808x over previous
import functools

import jax
import jax.numpy as jnp
from jax import lax
from jax.experimental import pallas as pl
from jax.experimental.pallas import tpu as pltpu
from jax.experimental.pallas import tpu_sc as plsc

_NC = 2
_NS = 16
_NW = _NC * _NS
_L = 16
_LN2 = 0.6931471805599453
_UNROLL = 5

_LC = (1.4720650106103683e-06, 0.9998476974962455, -0.49737321615804786,
       0.31574731675834483, -0.19035433673374297, 0.0826912371119928,
       -0.01741407752444414)
_LC5 = (2.211703119980868e-05, 0.9990104466294621, -0.48915684720231134,
        0.2833043245174014, -0.13011941539123476, 0.030102625011657738)


def _vlog(v):
    bits = lax.bitcast_convert_type(v, jnp.int32)
    k = jnp.right_shift(bits, 23) - 127
    mb = jnp.bitwise_or(jnp.bitwise_and(bits, 0x007FFFFF), 0x3F800000)
    f = lax.bitcast_convert_type(mb, jnp.float32) - 1.0
    p = _LC[6]
    for c in (_LC[5], _LC[4], _LC[3], _LC[2], _LC[1], _LC[0]):
        p = p * f + c
    return k.astype(jnp.float32) * _LN2 + p


def _sigmoid_terms(z):
    a = jnp.abs(z)
    t = jnp.exp(-a)
    q = 1.0 / (1.0 + t)
    pf = _LC5[5]
    for c in (_LC5[4], _LC5[3], _LC5[2], _LC5[1], _LC5[0]):
        pf = pf * t + c
    ent = pf + (1.0 - q) * a
    p = jnp.where(z < 0.0, 1.0 - q, q)
    return p, ent


def _sigmoid_only(z):
    zc = jnp.maximum(z, -80.0)
    return 1.0 / (1.0 + jnp.exp(-zc))


def _stage1_body(em_hbm, src_hbm, dst_hbm, x_hbm, nfm_hbm, w_hbm, out_hbm,
                 emv, dstv, srcv, hb, nfmv, wv, row1, accr, stage, tmpi,
                 sem, sma, smb, smc, smd, sme,
                 *, ch, d):
    wid = lax.axis_index("c") * _NS + lax.axis_index("s")
    base = wid * ch
    nvec = ch // _L
    ndg = d // _L
    cap = nvec

    c_em = pltpu.async_copy(em_hbm.at[pl.ds(base, ch)], emv, sma)
    c_dst = pltpu.async_copy(dst_hbm.at[pl.ds(base, ch)], dstv, smb)
    c_src = pltpu.async_copy(src_hbm.at[pl.ds(base, ch)], srcv, smc)
    c_nfm = pltpu.async_copy(nfm_hbm, nfmv, smd)
    c_w = pltpu.async_copy(w_hbm, wv, sme)

    for k in range(ndg):
        accr[pl.ds(k * _L, _L)] = jnp.zeros((_L,), jnp.float32)

    zf = jnp.zeros((_L,), jnp.float32)
    zi = jnp.zeros((_L,), jnp.int32)
    iota16 = lax.broadcasted_iota(jnp.int32, (_L,), 0)
    lane_base = iota16 * cap

    nun = nvec // _UNROLL

    c_em.wait()
    c_dst.wait()
    c_src.wait()

    def body(i, carry):
        cnt, s1s, s2s = carry
        s1s = list(s1s)
        s2s = list(s2s)
        for uu in range(_UNROLL):
            off = (i * _UNROLL + uu) * _L
            z = emv[pl.ds(off, _L)]
            p, ent = _sigmoid_terms(z)
            m = dstv[pl.ds(off, _L)] == 0
            slot = lane_base + cnt
            plsc.store_scatter(hb, [slot], off + iota16, mask=m)
            cnt = cnt + jnp.where(m, 1, 0)
            s1s[uu] = s1s[uu] + p
            s2s[uu] = s2s[uu] + ent
        return cnt, tuple(s1s), tuple(s2s)

    cnt, s1s, s2s = lax.fori_loop(
        0, nun, body, (zi, (zf,) * _UNROLL, (zf,) * _UNROLL))
    s1v = s1s[0] + s1s[1] + s1s[2] + s1s[3] + s1s[4]
    s2v = s2s[0] + s2s[1] + s2s[2] + s2s[3] + s2s[4]

    def dbody_for(l):
        def dbody(t, c):
            pos = jnp.full((_L,), l * cap + t, jnp.int32)
            ev = plsc.load_gather(hb, [pos])
            tmpi[...] = plsc.load_gather(srcv, [ev])
            zv = plsc.load_gather(emv, [ev])
            pltpu.async_copy(x_hbm.at[tmpi.at[pl.ds(0, 1)]], row1, sem).wait()
            wvv = _sigmoid_only(zv)
            for k in range(ndg):
                plsc.addupdate(accr.at[pl.ds(k * _L, _L)],
                               wvv * row1[0, pl.ds(k * _L, _L)])
            return c
        return dbody

    for l in range(_L):
        lax.fori_loop(0, cnt[l], dbody_for(l), jnp.int32(0))

    c_nfm.wait()
    c_w.wait()
    logits = zf
    for k in range(ndg):
        fmk = _sigmoid_only(nfmv[pl.ds(k * _L, _L)])
        aggk = accr[pl.ds(k * _L, _L)] * fmk
        for j in range(_L):
            logits = logits + aggk[j] * wv[k * _L + j]

    stage[pl.ds(0, _L)] = s1v
    stage[pl.ds(_L, _L)] = s2v
    stage[pl.ds(2 * _L, _L)] = logits
    pltpu.sync_copy(stage, out_hbm.at[wid])


def _epilogue_body(part_ref, nfm_ref, lab_ref, out_ref, *, e, d):
    eps = 1e-15
    part = part_ref[...]
    s1 = jnp.sum(part[:, 0:_L])
    s2 = jnp.sum(part[:, _L:2 * _L])
    logits = jnp.sum(part[:, 2 * _L:3 * _L], axis=0)

    nfm = nfm_ref[...]
    fm = 1.0 / (1.0 + jnp.exp(-nfm))
    ent2 = -fm * jnp.log(fm + eps) - (1.0 - fm) * jnp.log(1.0 - fm + eps)

    mx = jnp.max(logits)
    sh = logits - mx
    logsm = sh - jnp.log(jnp.sum(jnp.exp(sh)))
    lab = lab_ref[0, 0]
    ii = lax.broadcasted_iota(jnp.int32, (_L,), 0)
    pick = jnp.sum(jnp.where(ii == lab, logsm, 0.0))

    loss = (-pick
            + 0.005 * s1
            + s2 * (1.0 / float(e))
            + jnp.sum(fm)
            + jnp.sum(ent2) * (0.1 / float(d)))
    out_ref[...] = jnp.full((1, 1), loss, jnp.float32)


def kernel(x, edge_index, pred_label, node_feat_mask, edge_mask, W):
    e = edge_mask.shape[0]
    d = x.shape[1]
    c = W.shape[1]
    ch = e // _NW

    mesh = plsc.VectorSubcoreMesh(core_axis_name="c", subcore_axis_name="s",
                                  num_cores=_NC, num_subcores=_NS)
    f32 = jnp.float32
    params = pltpu.CompilerParams(needs_layout_passes=False)

    stage1 = pl.kernel(
        functools.partial(_stage1_body, ch=ch, d=d),
        out_type=jax.ShapeDtypeStruct((_NW, 3 * _L), f32),
        mesh=mesh,
        compiler_params=params,
        scratch_types=[
            pltpu.VMEM((ch,), f32),
            pltpu.VMEM((ch,), jnp.int32),
            pltpu.VMEM((ch,), jnp.int32),
            pltpu.VMEM((ch,), jnp.int32),
            pltpu.VMEM((d,), f32),
            pltpu.VMEM((d, c), f32),
            pltpu.VMEM((1, d), f32),
            pltpu.VMEM((d,), f32),
            pltpu.VMEM((3 * _L,), f32),
            pltpu.VMEM((_L,), jnp.int32),
            pltpu.SemaphoreType.DMA,
            pltpu.SemaphoreType.DMA,
            pltpu.SemaphoreType.DMA,
            pltpu.SemaphoreType.DMA,
            pltpu.SemaphoreType.DMA,
            pltpu.SemaphoreType.DMA,
        ],
    )
    part = stage1(edge_mask, edge_index[0], edge_index[1], x,
                  node_feat_mask, W)

    epilogue = pl.pallas_call(
        functools.partial(_epilogue_body, e=e, d=d),
        out_shape=jax.ShapeDtypeStruct((1, 1), f32),
    )
    out = epilogue(part, node_feat_mask.reshape(1, d),
                   pred_label[:1].reshape(1, 1))
    return out[0, 0]


# speedup vs baseline: 41.2806x; 1.0010x over previous
import functools

import jax
import jax.numpy as jnp
from jax import lax
from jax.experimental import pallas as pl
from jax.experimental.pallas import tpu as pltpu
from jax.experimental.pallas import tpu_sc as plsc

_NC = 2
_NS = 16
_NW = _NC * _NS
_L = 16
_LN2 = 0.6931471805599453
_UNROLL = 5

_LC = (1.4720650106103683e-06, 0.9998476974962455, -0.49737321615804786,
       0.31574731675834483, -0.19035433673374297, 0.0826912371119928,
       -0.01741407752444414)
_LC5 = (0.0001415121753789439, 0.9954273382579881, -0.4640725804471214,
        0.21641043832781495, -0.05486285286206372)


def _vlog(v):
    bits = lax.bitcast_convert_type(v, jnp.int32)
    k = jnp.right_shift(bits, 23) - 127
    mb = jnp.bitwise_or(jnp.bitwise_and(bits, 0x007FFFFF), 0x3F800000)
    f = lax.bitcast_convert_type(mb, jnp.float32) - 1.0
    p = _LC[6]
    for c in (_LC[5], _LC[4], _LC[3], _LC[2], _LC[1], _LC[0]):
        p = p * f + c
    return k.astype(jnp.float32) * _LN2 + p


def _sigmoid_terms(z):
    a = jnp.abs(z)
    t = jnp.exp(-a)
    q = 1.0 / (1.0 + t)
    pf = _LC5[4]
    for c in (_LC5[3], _LC5[2], _LC5[1], _LC5[0]):
        pf = pf * t + c
    ent = pf + (1.0 - q) * a
    p = jnp.where(z < 0.0, 1.0 - q, q)
    return p, ent


def _sigmoid_only(z):
    zc = jnp.maximum(z, -80.0)
    return 1.0 / (1.0 + jnp.exp(-zc))


def _stage1_body(em_hbm, src_hbm, dst_hbm, x_hbm, nfm_hbm, w_hbm, out_hbm,
                 emv, dstv, srcv, hb, nfmv, wv, row1, accr, stage, tmpi,
                 sem, sma, smb, smc, smd, sme,
                 *, ch, d):
    wid = lax.axis_index("c") * _NS + lax.axis_index("s")
    base = wid * ch
    nvec = ch // _L
    ndg = d // _L
    cap = nvec

    c_em = pltpu.async_copy(em_hbm.at[pl.ds(base, ch)], emv, sma)
    c_dst = pltpu.async_copy(dst_hbm.at[pl.ds(base, ch)], dstv, smb)
    c_src = pltpu.async_copy(src_hbm.at[pl.ds(base, ch)], srcv, smc)
    c_nfm = pltpu.async_copy(nfm_hbm, nfmv, smd)
    c_w = pltpu.async_copy(w_hbm, wv, sme)

    for k in range(ndg):
        accr[pl.ds(k * _L, _L)] = jnp.zeros((_L,), jnp.float32)

    zf = jnp.zeros((_L,), jnp.float32)
    zi = jnp.zeros((_L,), jnp.int32)
    iota16 = lax.broadcasted_iota(jnp.int32, (_L,), 0)
    lane_base = iota16 * cap

    nun = nvec // _UNROLL

    c_em.wait()
    c_dst.wait()
    c_src.wait()

    def body(i, carry):
        cnt, s1s, s2s = carry
        s1s = list(s1s)
        s2s = list(s2s)
        for uu in range(_UNROLL):
            off = (i * _UNROLL + uu) * _L
            z = emv[pl.ds(off, _L)]
            p, ent = _sigmoid_terms(z)
            m = dstv[pl.ds(off, _L)] == 0
            slot = lane_base + cnt
            plsc.store_scatter(hb, [slot], off + iota16, mask=m)
            cnt = cnt + jnp.where(m, 1, 0)
            s1s[uu] = s1s[uu] + p
            s2s[uu] = s2s[uu] + ent
        return cnt, tuple(s1s), tuple(s2s)

    cnt, s1s, s2s = lax.fori_loop(
        0, nun, body, (zi, (zf,) * _UNROLL, (zf,) * _UNROLL))
    s1v = s1s[0] + s1s[1] + s1s[2] + s1s[3] + s1s[4]
    s2v = s2s[0] + s2s[1] + s2s[2] + s2s[3] + s2s[4]

    def dbody_for(l):
        def dbody(t, c):
            pos = jnp.full((_L,), l * cap + t, jnp.int32)
            ev = plsc.load_gather(hb, [pos])
            tmpi[...] = plsc.load_gather(srcv, [ev])
            zv = plsc.load_gather(emv, [ev])
            pltpu.async_copy(x_hbm.at[tmpi.at[pl.ds(0, 1)]], row1, sem).wait()
            wvv = _sigmoid_only(zv)
            for k in range(ndg):
                plsc.addupdate(accr.at[pl.ds(k * _L, _L)],
                               wvv * row1[0, pl.ds(k * _L, _L)])
            return c
        return dbody

    for l in range(_L):
        lax.fori_loop(0, cnt[l], dbody_for(l), jnp.int32(0))

    c_nfm.wait()
    c_w.wait()
    logits = zf
    for k in range(ndg):
        fmk = _sigmoid_only(nfmv[pl.ds(k * _L, _L)])
        aggk = accr[pl.ds(k * _L, _L)] * fmk
        for j in range(_L):
            logits = logits + aggk[j] * wv[k * _L + j]

    stage[pl.ds(0, _L)] = s1v
    stage[pl.ds(_L, _L)] = s2v
    stage[pl.ds(2 * _L, _L)] = logits
    pltpu.sync_copy(stage, out_hbm.at[wid])


def _epilogue_body(part_ref, nfm_ref, lab_ref, out_ref, *, e, d):
    eps = 1e-15
    part = part_ref[...]
    s1 = jnp.sum(part[:, 0:_L])
    s2 = jnp.sum(part[:, _L:2 * _L])
    logits = jnp.sum(part[:, 2 * _L:3 * _L], axis=0)

    nfm = nfm_ref[...]
    fm = 1.0 / (1.0 + jnp.exp(-nfm))
    ent2 = -fm * jnp.log(fm + eps) - (1.0 - fm) * jnp.log(1.0 - fm + eps)

    mx = jnp.max(logits)
    sh = logits - mx
    logsm = sh - jnp.log(jnp.sum(jnp.exp(sh)))
    lab = lab_ref[0, 0]
    ii = lax.broadcasted_iota(jnp.int32, (_L,), 0)
    pick = jnp.sum(jnp.where(ii == lab, logsm, 0.0))

    loss = (-pick
            + 0.005 * s1
            + s2 * (1.0 / float(e))
            + jnp.sum(fm)
            + jnp.sum(ent2) * (0.1 / float(d)))
    out_ref[...] = jnp.full((1, 1), loss, jnp.float32)


def kernel(x, edge_index, pred_label, node_feat_mask, edge_mask, W):
    e = edge_mask.shape[0]
    d = x.shape[1]
    c = W.shape[1]
    ch = e // _NW

    mesh = plsc.VectorSubcoreMesh(core_axis_name="c", subcore_axis_name="s",
                                  num_cores=_NC, num_subcores=_NS)
    f32 = jnp.float32
    params = pltpu.CompilerParams(needs_layout_passes=False)

    stage1 = pl.kernel(
        functools.partial(_stage1_body, ch=ch, d=d),
        out_type=jax.ShapeDtypeStruct((_NW, 3 * _L), f32),
        mesh=mesh,
        compiler_params=params,
        scratch_types=[
            pltpu.VMEM((ch,), f32),
            pltpu.VMEM((ch,), jnp.int32),
            pltpu.VMEM((ch,), jnp.int32),
            pltpu.VMEM((ch,), jnp.int32),
            pltpu.VMEM((d,), f32),
            pltpu.VMEM((d, c), f32),
            pltpu.VMEM((1, d), f32),
            pltpu.VMEM((d,), f32),
            pltpu.VMEM((3 * _L,), f32),
            pltpu.VMEM((_L,), jnp.int32),
            pltpu.SemaphoreType.DMA,
            pltpu.SemaphoreType.DMA,
            pltpu.SemaphoreType.DMA,
            pltpu.SemaphoreType.DMA,
            pltpu.SemaphoreType.DMA,
            pltpu.SemaphoreType.DMA,
        ],
    )
    part = stage1(edge_mask, edge_index[0], edge_index[1], x,
                  node_feat_mask, W)

    epilogue = pl.pallas_call(
        functools.partial(_epilogue_body, e=e, d=d),
        out_shape=jax.ShapeDtypeStruct((1, 1), f32),
    )
    out = epilogue(part, node_feat_mask.reshape(1, d),
                   pred_label[:1].reshape(1, 1))
    return out[0, 0]
